# Initial kernel scaffold; baseline (speedup 1.0000x reference)
import jax
import jax.numpy as jnp
from jax.experimental import pallas as pl


def kernel(x_src, x_dst, edge_attr, edge_index, W_src, W_edge):
    raise NotImplementedError("write your pallas kernel here")


# speedup vs baseline: 3.0947x; 3.0947x over previous
import functools

import jax
import jax.numpy as jnp
from jax import lax
from jax.experimental import pallas as pl
from jax.experimental.pallas import tpu as pltpu
from jax.experimental.pallas import tpu_sc as plsc

N_SRC = 10000
N_DST = 10000
E = 320000
D_FEAT = 128
D_EDGE = 16

NC = 2
NS = 16
NW = NC * NS
CH = 128
NCHUNK = E // CH
JMAX = (NCHUNK + NW - 1) // NW
RCH = 128
NRCH = (N_DST + RCH - 1) // RCH
RTAIL = N_DST - (NRCH - 1) * RCH


def _sc_agg(y_hbm, m_hbm, sidx_hbm, didx_hbm, zg_hbm,
            t_out,
            sidx_v, didx_v, rows_v, m_v, t_sh, sem):
    cid = lax.axis_index("c")
    sid = lax.axis_index("s")
    w = sid * NC + cid

    pltpu.sync_copy(zg_hbm, rows_v)

    def zbody(j, carry):
        k = j * NS + sid
        r0 = k * RCH

        @pl.when(k < NRCH - 1)
        def _():
            pltpu.sync_copy(rows_v, t_sh.at[pl.ds(r0, RCH)])

        @pl.when(k == NRCH - 1)
        def _():
            pltpu.sync_copy(rows_v.at[pl.ds(0, RTAIL)],
                            t_sh.at[pl.ds(r0, RTAIL)])

        return carry

    lax.fori_loop(0, (NRCH + NS - 1) // NS, zbody, 0)
    plsc.subcore_barrier()

    def body(j, carry):
        c = j * NW + w

        @pl.when(c < NCHUNK)
        def _():
            e0 = c * CH
            pltpu.sync_copy(sidx_hbm.at[pl.ds(e0, CH)], sidx_v)
            pltpu.sync_copy(didx_hbm.at[pl.ds(e0, CH)], didx_v)
            pltpu.sync_copy(m_hbm.at[pl.ds(e0, CH)], m_v)
            pltpu.async_copy(y_hbm.at[sidx_v], rows_v, sem).wait()
            pltpu.sync_copy(rows_v, t_sh.at[didx_v], add=True)
            pltpu.sync_copy(m_v, t_sh.at[didx_v], add=True)

        return carry

    lax.fori_loop(0, JMAX, body, 0)
    plsc.subcore_barrier()

    def obody(j, carry):
        k = j * NS + sid
        r0 = k * RCH
        ob = cid * N_DST + r0

        @pl.when(k < NRCH - 1)
        def _():
            pltpu.sync_copy(t_sh.at[pl.ds(r0, RCH)], rows_v)
            pltpu.sync_copy(rows_v, t_out.at[pl.ds(ob, RCH)])

        @pl.when(k == NRCH - 1)
        def _():
            pltpu.sync_copy(t_sh.at[pl.ds(r0, RTAIL)],
                            rows_v.at[pl.ds(0, RTAIL)])
            pltpu.sync_copy(rows_v.at[pl.ds(0, RTAIL)],
                            t_out.at[pl.ds(ob, RTAIL)])

        return carry

    lax.fori_loop(0, (NRCH + NS - 1) // NS, obody, 0)


_sc_agg_call = functools.partial(
    pl.kernel,
    out_type=jax.ShapeDtypeStruct((NC * N_DST, D_FEAT), jnp.float32),
    mesh=plsc.VectorSubcoreMesh(core_axis_name="c", subcore_axis_name="s"),
    scratch_types=[
        pltpu.VMEM((CH,), jnp.int32),
        pltpu.VMEM((CH,), jnp.int32),
        pltpu.VMEM((CH, D_FEAT), jnp.float32),
        pltpu.VMEM((CH, D_FEAT), jnp.float32),
        pltpu.VMEM_SHARED((N_DST, D_FEAT), jnp.float32),
        pltpu.SemaphoreType.DMA,
    ],
)(_sc_agg)


def _y_body(xs, ws, y):
    y[...] = jnp.dot(xs[...], ws[...], preferred_element_type=jnp.float32)


def _m_body(ea, we, m):
    m[...] = jnp.dot(ea[...], we[...], preferred_element_type=jnp.float32)


def _out_body(xd, t0, t1, out):
    out[...] = xd[...] + t0[...] + t1[...]


_BM = 1000
_BE = 4000


def _tc_pre(x_src, edge_attr, W_src, W_edge):
    y = pl.pallas_call(
        _y_body,
        grid=(N_SRC // _BM,),
        in_specs=[
            pl.BlockSpec((_BM, D_FEAT), lambda i: (i, 0)),
            pl.BlockSpec((D_FEAT, D_FEAT), lambda i: (0, 0)),
        ],
        out_specs=pl.BlockSpec((_BM, D_FEAT), lambda i: (i, 0)),
        out_shape=jax.ShapeDtypeStruct((N_SRC, D_FEAT), jnp.float32),
    )(x_src, W_src)
    m = pl.pallas_call(
        _m_body,
        grid=(E // _BE,),
        in_specs=[
            pl.BlockSpec((_BE, D_EDGE), lambda i: (i, 0)),
            pl.BlockSpec((D_EDGE, D_FEAT), lambda i: (0, 0)),
        ],
        out_specs=pl.BlockSpec((_BE, D_FEAT), lambda i: (i, 0)),
        out_shape=jax.ShapeDtypeStruct((E, D_FEAT), jnp.float32),
    )(edge_attr, W_edge)
    return y, m


def _tc_post(x_dst, t0, t1):
    return pl.pallas_call(
        _out_body,
        grid=(N_DST // _BM,),
        in_specs=[
            pl.BlockSpec((_BM, D_FEAT), lambda i: (i, 0)),
            pl.BlockSpec((_BM, D_FEAT), lambda i: (i, 0)),
            pl.BlockSpec((_BM, D_FEAT), lambda i: (i, 0)),
        ],
        out_specs=pl.BlockSpec((_BM, D_FEAT), lambda i: (i, 0)),
        out_shape=jax.ShapeDtypeStruct((N_DST, D_FEAT), jnp.float32),
    )(x_dst, t0, t1)


def kernel(x_src, x_dst, edge_attr, edge_index, W_src, W_edge):
    src = edge_index[0].astype(jnp.int32)
    dst = edge_index[1].astype(jnp.int32)
    zg = jnp.zeros((CH, D_FEAT), jnp.float32)

    y, m = _tc_pre(x_src, edge_attr, W_src, W_edge)
    t = _sc_agg_call(y, m, src, dst, zg)
    return _tc_post(x_dst, t[:N_DST], t[N_DST:])


# speedup vs baseline: 4.0979x; 1.3242x over previous
import functools

import jax
import jax.numpy as jnp
from jax import lax
from jax.experimental import pallas as pl
from jax.experimental.pallas import tpu as pltpu
from jax.experimental.pallas import tpu_sc as plsc

N_SRC = 10000
N_DST = 10000
E = 320000
D_FEAT = 128
D_EDGE = 16

NC = 2
NS = 16
NW = NC * NS
CH = 128
NCHUNK = E // CH
JMAX = (NCHUNK + NW - 1) // NW
RCH = 128
NRCH = (N_DST + RCH - 1) // RCH
RTAIL = N_DST - (NRCH - 1) * RCH


NFULL = NCHUNK // NW * NW
JFULL = NCHUNK // NW


def _sc_agg(y_hbm, m_hbm, sidx_hbm, didx_hbm, zg_hbm,
            t_out,
            sidx0, didx0, rows0,
            sidx1, didx1, rows1,
            t_sh, sem, sem_l0, sem_l1, sem_s0, sem_s1):
    cid = lax.axis_index("c")
    sid = lax.axis_index("s")
    w = sid * NC + cid

    sidx_v, didx_v, rows_v = sidx0, didx0, rows0
    bufs = ((sidx0, didx0, rows0, sem_l0, sem_s0),
            (sidx1, didx1, rows1, sem_l1, sem_s1))

    pltpu.sync_copy(zg_hbm, rows_v)

    def zbody(j, carry):
        k = j * NS + sid
        r0 = k * RCH

        @pl.when(k < NRCH - 1)
        def _():
            pltpu.sync_copy(rows_v, t_sh.at[pl.ds(r0, RCH)])

        @pl.when(k == NRCH - 1)
        def _():
            pltpu.sync_copy(rows_v.at[pl.ds(0, RTAIL)],
                            t_sh.at[pl.ds(r0, RTAIL)])

        return carry

    lax.fori_loop(0, (NRCH + NS - 1) // NS, zbody, 0)
    plsc.subcore_barrier()

    def issue_loads_m(buf, c):
        _, dv, rv, sl, _ = buf
        e0 = c * CH
        pltpu.async_copy(didx_hbm.at[pl.ds(e0, CH)], dv, sl)
        pltpu.async_copy(m_hbm.at[pl.ds(e0, CH)], rv, sl)

    def wait_loads_m(buf, c):
        _, dv, rv, sl, _ = buf
        e0 = c * CH
        pltpu.make_async_copy(didx_hbm.at[pl.ds(e0, CH)], dv, sl).wait()
        pltpu.make_async_copy(m_hbm.at[pl.ds(e0, CH)], rv, sl).wait()

    def issue_loads_y(buf, c):
        sv, dv, _, sl, _ = buf
        e0 = c * CH
        pltpu.async_copy(sidx_hbm.at[pl.ds(e0, CH)], sv, sl)
        pltpu.async_copy(didx_hbm.at[pl.ds(e0, CH)], dv, sl)

    def wait_loads_y(buf, c):
        sv, dv, _, sl, _ = buf
        e0 = c * CH
        pltpu.make_async_copy(sidx_hbm.at[pl.ds(e0, CH)], sv, sl).wait()
        pltpu.make_async_copy(didx_hbm.at[pl.ds(e0, CH)], dv, sl).wait()

    def issue_scatter(buf):
        _, dv, rv, _, ss = buf
        pltpu.async_copy(rv, t_sh.at[dv], ss, add=True)

    def wait_scatter(buf):
        _, dv, rv, _, ss = buf
        pltpu.make_async_copy(rv, t_sh.at[dv], ss).wait()

    issue_loads_m(bufs[0], w)

    def mbody(u, carry):
        for b in (0, 1):
            buf = bufs[b]
            j = 2 * u + b
            c = j * NW + w

            @pl.when(j >= 2)
            def _():
                wait_scatter(buf)

            wait_loads_m(buf, c)

            @pl.when(j + 1 < JFULL)
            def _():
                issue_loads_m(bufs[1 - b], (j + 1) * NW + w)

            issue_scatter(buf)
        return carry

    lax.fori_loop(0, JFULL // 2, mbody, 0)
    wait_scatter(bufs[0])
    wait_scatter(bufs[1])

    issue_loads_y(bufs[0], w)

    def ybody(u, carry):
        for b in (0, 1):
            buf = bufs[b]
            j = 2 * u + b
            c = j * NW + w

            @pl.when(j >= 2)
            def _():
                wait_scatter(buf)

            wait_loads_y(buf, c)

            @pl.when(j + 1 < JFULL)
            def _():
                issue_loads_y(bufs[1 - b], (j + 1) * NW + w)

            sv, dv, rv, _, _ = buf
            pltpu.async_copy(y_hbm.at[sv], rv, sem).wait()
            issue_scatter(buf)
        return carry

    lax.fori_loop(0, JFULL // 2, ybody, 0)
    wait_scatter(bufs[0])
    wait_scatter(bufs[1])

    @pl.when(w < NCHUNK - NFULL)
    def _():
        c = NFULL + w
        e0 = c * CH
        pltpu.sync_copy(sidx_hbm.at[pl.ds(e0, CH)], sidx_v)
        pltpu.sync_copy(didx_hbm.at[pl.ds(e0, CH)], didx_v)
        pltpu.async_copy(y_hbm.at[sidx_v], rows_v, sem).wait()
        pltpu.sync_copy(rows_v, t_sh.at[didx_v], add=True)
        pltpu.sync_copy(m_hbm.at[pl.ds(e0, CH)], rows_v)
        pltpu.sync_copy(rows_v, t_sh.at[didx_v], add=True)

    plsc.subcore_barrier()

    def obody(j, carry):
        k = j * NS + sid
        r0 = k * RCH
        ob = cid * N_DST + r0

        @pl.when(k < NRCH - 1)
        def _():
            pltpu.sync_copy(t_sh.at[pl.ds(r0, RCH)], rows_v)
            pltpu.sync_copy(rows_v, t_out.at[pl.ds(ob, RCH)])

        @pl.when(k == NRCH - 1)
        def _():
            pltpu.sync_copy(t_sh.at[pl.ds(r0, RTAIL)],
                            rows_v.at[pl.ds(0, RTAIL)])
            pltpu.sync_copy(rows_v.at[pl.ds(0, RTAIL)],
                            t_out.at[pl.ds(ob, RTAIL)])

        return carry

    lax.fori_loop(0, (NRCH + NS - 1) // NS, obody, 0)


_sc_agg_call = functools.partial(
    pl.kernel,
    out_type=jax.ShapeDtypeStruct((NC * N_DST, D_FEAT), jnp.float32),
    mesh=plsc.VectorSubcoreMesh(core_axis_name="c", subcore_axis_name="s"),
    scratch_types=[
        pltpu.VMEM((CH,), jnp.int32),
        pltpu.VMEM((CH,), jnp.int32),
        pltpu.VMEM((CH, D_FEAT), jnp.float32),
        pltpu.VMEM((CH,), jnp.int32),
        pltpu.VMEM((CH,), jnp.int32),
        pltpu.VMEM((CH, D_FEAT), jnp.float32),
        pltpu.VMEM_SHARED((N_DST, D_FEAT), jnp.float32),
        pltpu.SemaphoreType.DMA,
        pltpu.SemaphoreType.DMA,
        pltpu.SemaphoreType.DMA,
        pltpu.SemaphoreType.DMA,
        pltpu.SemaphoreType.DMA,
    ],
)(_sc_agg)


def _y_body(xs, ws, y):
    y[...] = jnp.dot(xs[...], ws[...], preferred_element_type=jnp.float32)


def _m_body(ea, we, m):
    m[...] = jnp.dot(ea[...], we[...], preferred_element_type=jnp.float32)


def _out_body(xd, t0, t1, out):
    out[...] = xd[...] + t0[...] + t1[...]


_BM = 1000
_BE = 4000


def _tc_pre(x_src, edge_attr, W_src, W_edge):
    y = pl.pallas_call(
        _y_body,
        grid=(N_SRC // _BM,),
        in_specs=[
            pl.BlockSpec((_BM, D_FEAT), lambda i: (i, 0)),
            pl.BlockSpec((D_FEAT, D_FEAT), lambda i: (0, 0)),
        ],
        out_specs=pl.BlockSpec((_BM, D_FEAT), lambda i: (i, 0)),
        out_shape=jax.ShapeDtypeStruct((N_SRC, D_FEAT), jnp.float32),
    )(x_src, W_src)
    m = pl.pallas_call(
        _m_body,
        grid=(E // _BE,),
        in_specs=[
            pl.BlockSpec((_BE, D_EDGE), lambda i: (i, 0)),
            pl.BlockSpec((D_EDGE, D_FEAT), lambda i: (0, 0)),
        ],
        out_specs=pl.BlockSpec((_BE, D_FEAT), lambda i: (i, 0)),
        out_shape=jax.ShapeDtypeStruct((E, D_FEAT), jnp.float32),
    )(edge_attr, W_edge)
    return y, m


def _tc_post(x_dst, t0, t1):
    return pl.pallas_call(
        _out_body,
        grid=(N_DST // _BM,),
        in_specs=[
            pl.BlockSpec((_BM, D_FEAT), lambda i: (i, 0)),
            pl.BlockSpec((_BM, D_FEAT), lambda i: (i, 0)),
            pl.BlockSpec((_BM, D_FEAT), lambda i: (i, 0)),
        ],
        out_specs=pl.BlockSpec((_BM, D_FEAT), lambda i: (i, 0)),
        out_shape=jax.ShapeDtypeStruct((N_DST, D_FEAT), jnp.float32),
    )(x_dst, t0, t1)


def kernel(x_src, x_dst, edge_attr, edge_index, W_src, W_edge):
    src = edge_index[0].astype(jnp.int32)
    dst = edge_index[1].astype(jnp.int32)
    zg = jnp.zeros((CH, D_FEAT), jnp.float32)

    y, m = _tc_pre(x_src, edge_attr, W_src, W_edge)
    t = _sc_agg_call(y, m, src, dst, zg)
    return _tc_post(x_dst, t[:N_DST], t[N_DST:])


# speedup vs baseline: 5.0581x; 1.2343x over previous
import functools

import jax
import jax.numpy as jnp
from jax import lax
from jax.experimental import pallas as pl
from jax.experimental.pallas import tpu as pltpu
from jax.experimental.pallas import tpu_sc as plsc

N_SRC = 10000
N_DST = 10000
E = 320000
D_FEAT = 128
D_EDGE = 16

NC = 2
NS = 16
NW = NC * NS
CH = 128
NCHUNK = E // CH
JMAX = (NCHUNK + NW - 1) // NW
RCH = 128
NRCH = (N_DST + RCH - 1) // RCH
RTAIL = N_DST - (NRCH - 1) * RCH


NFULL = NCHUNK // NW * NW
JFULL = NCHUNK // NW


def _sc_agg(y_hbm, m_hbm, sidx_hbm, didx_hbm, zg_hbm,
            t_out,
            sidx0, didx0, rows0,
            sidx1, didx1, rows1,
            t_sh, sem, sem_l0, sem_l1, sem_s0, sem_s1):
    cid = lax.axis_index("c")
    sid = lax.axis_index("s")
    w = sid * NC + cid

    sidx_v, didx_v, rows_v = sidx0, didx0, rows0
    bufs = ((sidx0, didx0, rows0, sem_l0, sem_s0),
            (sidx1, didx1, rows1, sem_l1, sem_s1))

    pltpu.sync_copy(zg_hbm, rows_v)

    def zbody(j, carry):
        k = j * NS + sid
        r0 = k * RCH

        @pl.when(k < NRCH - 1)
        def _():
            pltpu.sync_copy(rows_v, t_sh.at[pl.ds(r0, RCH)])

        @pl.when(k == NRCH - 1)
        def _():
            pltpu.sync_copy(rows_v.at[pl.ds(0, RTAIL)],
                            t_sh.at[pl.ds(r0, RTAIL)])

        return carry

    lax.fori_loop(0, (NRCH + NS - 1) // NS, zbody, 0)
    plsc.subcore_barrier()

    def issue_loads_m(buf, c):
        _, dv, rv, sl, _ = buf
        e0 = c * CH
        pltpu.async_copy(didx_hbm.at[pl.ds(e0, CH)], dv, sl)
        pltpu.async_copy(m_hbm.at[pl.ds(e0, CH)], rv, sl)

    def wait_loads_m(buf, c):
        _, dv, rv, sl, _ = buf
        e0 = c * CH
        pltpu.make_async_copy(didx_hbm.at[pl.ds(e0, CH)], dv, sl).wait()
        pltpu.make_async_copy(m_hbm.at[pl.ds(e0, CH)], rv, sl).wait()

    def issue_loads_y(buf, c):
        sv, dv, _, sl, _ = buf
        e0 = c * CH
        pltpu.async_copy(sidx_hbm.at[pl.ds(e0, CH)], sv, sl)
        pltpu.async_copy(didx_hbm.at[pl.ds(e0, CH)], dv, sl)

    def wait_loads_y(buf, c):
        sv, dv, _, sl, _ = buf
        e0 = c * CH
        pltpu.make_async_copy(sidx_hbm.at[pl.ds(e0, CH)], sv, sl).wait()
        pltpu.make_async_copy(didx_hbm.at[pl.ds(e0, CH)], dv, sl).wait()

    def issue_scatter(buf):
        _, dv, rv, _, ss = buf
        pltpu.async_copy(rv, t_sh.at[dv], ss, add=True)

    def wait_scatter(buf):
        _, dv, rv, _, ss = buf
        pltpu.make_async_copy(rv, t_sh.at[dv], ss).wait()

    issue_loads_m(bufs[0], w)

    def mbody(u, carry):
        for b in (0, 1):
            buf = bufs[b]
            j = 2 * u + b
            c = j * NW + w

            @pl.when(j >= 2)
            def _():
                wait_scatter(buf)

            wait_loads_m(buf, c)

            @pl.when(j + 1 < JFULL)
            def _():
                issue_loads_m(bufs[1 - b], (j + 1) * NW + w)

            issue_scatter(buf)
        return carry

    lax.fori_loop(0, JFULL // 2, mbody, 0)
    wait_scatter(bufs[0])
    wait_scatter(bufs[1])

    issue_loads_y(bufs[0], w)

    def ybody(u, carry):
        for b in (0, 1):
            buf = bufs[b]
            j = 2 * u + b
            c = j * NW + w

            @pl.when(j >= 2)
            def _():
                wait_scatter(buf)

            wait_loads_y(buf, c)

            @pl.when(j + 1 < JFULL)
            def _():
                issue_loads_y(bufs[1 - b], (j + 1) * NW + w)

            sv, dv, rv, _, _ = buf
            pltpu.async_copy(y_hbm.at[sv], rv, sem).wait()
            issue_scatter(buf)
        return carry

    lax.fori_loop(0, JFULL // 2, ybody, 0)
    wait_scatter(bufs[0])
    wait_scatter(bufs[1])

    @pl.when(w < NCHUNK - NFULL)
    def _():
        c = NFULL + w
        e0 = c * CH
        pltpu.sync_copy(sidx_hbm.at[pl.ds(e0, CH)], sidx_v)
        pltpu.sync_copy(didx_hbm.at[pl.ds(e0, CH)], didx_v)
        pltpu.async_copy(y_hbm.at[sidx_v], rows_v, sem).wait()
        pltpu.sync_copy(rows_v, t_sh.at[didx_v], add=True)
        pltpu.sync_copy(m_hbm.at[pl.ds(e0, CH)], rows_v)
        pltpu.sync_copy(rows_v, t_sh.at[didx_v], add=True)

    plsc.subcore_barrier()

    def obody(j, carry):
        k = j * NS + sid
        r0 = k * RCH
        ob = cid * N_DST + r0

        @pl.when(k < NRCH - 1)
        def _():
            pltpu.sync_copy(t_sh.at[pl.ds(r0, RCH)], rows_v)
            pltpu.sync_copy(rows_v, t_out.at[pl.ds(ob, RCH)])

        @pl.when(k == NRCH - 1)
        def _():
            pltpu.sync_copy(t_sh.at[pl.ds(r0, RTAIL)],
                            rows_v.at[pl.ds(0, RTAIL)])
            pltpu.sync_copy(rows_v.at[pl.ds(0, RTAIL)],
                            t_out.at[pl.ds(ob, RTAIL)])

        return carry

    lax.fori_loop(0, (NRCH + NS - 1) // NS, obody, 0)


_sc_agg_call = functools.partial(
    pl.kernel,
    out_type=jax.ShapeDtypeStruct((NC * N_DST, D_FEAT), jnp.float32),
    mesh=plsc.VectorSubcoreMesh(core_axis_name="c", subcore_axis_name="s"),
    scratch_types=[
        pltpu.VMEM((CH,), jnp.int32),
        pltpu.VMEM((CH,), jnp.int32),
        pltpu.VMEM((CH, D_FEAT), jnp.float32),
        pltpu.VMEM((CH,), jnp.int32),
        pltpu.VMEM((CH,), jnp.int32),
        pltpu.VMEM((CH, D_FEAT), jnp.float32),
        pltpu.VMEM_SHARED((N_DST, D_FEAT), jnp.float32),
        pltpu.SemaphoreType.DMA,
        pltpu.SemaphoreType.DMA,
        pltpu.SemaphoreType.DMA,
        pltpu.SemaphoreType.DMA,
        pltpu.SemaphoreType.DMA,
    ],
)(_sc_agg)


def _y_body(xs, ws, y):
    y[...] = jnp.dot(xs[...], ws[...], preferred_element_type=jnp.float32)


def _m_body(eat, we, m):
    m[...] = lax.dot_general(
        eat[...], we[...], (((0,), (0,)), ((), ())),
        preferred_element_type=jnp.float32)


def _out_body(xd, t0, t1, out):
    out[...] = xd[...] + t0[...] + t1[...]


_BM = 1000
_BE = 3200


def _tc_pre(x_src, edge_attr, W_src, W_edge):
    y = pl.pallas_call(
        _y_body,
        grid=(N_SRC // _BM,),
        in_specs=[
            pl.BlockSpec((_BM, D_FEAT), lambda i: (i, 0)),
            pl.BlockSpec((D_FEAT, D_FEAT), lambda i: (0, 0)),
        ],
        out_specs=pl.BlockSpec((_BM, D_FEAT), lambda i: (i, 0)),
        out_shape=jax.ShapeDtypeStruct((N_SRC, D_FEAT), jnp.float32),
    )(x_src, W_src)
    m = pl.pallas_call(
        _m_body,
        grid=(E // _BE,),
        in_specs=[
            pl.BlockSpec((D_EDGE, _BE), lambda i: (0, i)),
            pl.BlockSpec((D_EDGE, D_FEAT), lambda i: (0, 0)),
        ],
        out_specs=pl.BlockSpec((_BE, D_FEAT), lambda i: (i, 0)),
        out_shape=jax.ShapeDtypeStruct((E, D_FEAT), jnp.float32),
    )(edge_attr.T, W_edge)
    return y, m


def _tc_post(x_dst, t0, t1):
    return pl.pallas_call(
        _out_body,
        grid=(N_DST // _BM,),
        in_specs=[
            pl.BlockSpec((_BM, D_FEAT), lambda i: (i, 0)),
            pl.BlockSpec((_BM, D_FEAT), lambda i: (i, 0)),
            pl.BlockSpec((_BM, D_FEAT), lambda i: (i, 0)),
        ],
        out_specs=pl.BlockSpec((_BM, D_FEAT), lambda i: (i, 0)),
        out_shape=jax.ShapeDtypeStruct((N_DST, D_FEAT), jnp.float32),
    )(x_dst, t0, t1)


def kernel(x_src, x_dst, edge_attr, edge_index, W_src, W_edge):
    src = edge_index[0].astype(jnp.int32)
    dst = edge_index[1].astype(jnp.int32)
    zg = jnp.zeros((CH, D_FEAT), jnp.float32)

    y, m = _tc_pre(x_src, edge_attr, W_src, W_edge)
    t = _sc_agg_call(y, m, src, dst, zg)
    return _tc_post(x_dst, t[:N_DST], t[N_DST:])
